# Initial kernel scaffold; baseline (speedup 1.0000x reference)
import jax
import jax.numpy as jnp
from jax.experimental import pallas as pl


def kernel(w1, b1, w2, b2, fc1_w, fc1_b, fc2_w, fc2_b, fc3_w, fc3_b, x):
    raise NotImplementedError("write your pallas kernel here")


# speedup vs baseline: 448.5977x; 448.5977x over previous
import functools

import jax
import jax.numpy as jnp
from jax.experimental import pallas as pl
from jax.experimental.pallas import tpu as pltpu


def _toeplitz_band(v, width, rows):
    k = v.shape[-1]
    period = width + 1
    vp = jnp.pad(v, [(0, 0)] * (v.ndim - 1) + [(0, period - k)])
    t = jnp.tile(vp, (1,) * (v.ndim - 1) + (rows,))[..., : rows * width]
    return t.reshape(v.shape[:-1] + (rows, width))


def _fused_cnn_kernel(x_ref, t1_ref, b1_ref, t2_ref, b2_ref,
                      w1_ref, bf1_ref, w2_ref, bf2_ref, w3_ref, bf3_ref,
                      o_ref, a1_ref, a2_ref, *, nb):
    f32 = jnp.float32
    bf16 = jnp.bfloat16

    t1 = t1_ref[...]
    for r in range(14):
        y0 = 2 * r
        s0 = x_ref[:, y0:y0 + 5, :, :].reshape(480, nb)
        s1 = x_ref[:, y0 + 1:y0 + 6, :, :].reshape(480, nb)
        o0 = jnp.dot(t1, s0, preferred_element_type=f32)
        o1 = jnp.dot(t1, s1, preferred_element_type=f32)
        m = jnp.maximum(o0, o1) + b1_ref[...]
        m = jnp.maximum(m, 0.0)
        v = m.reshape(14, 2, 8, nb)
        a1_ref[r] = jnp.maximum(v[:, 0], v[:, 1]).astype(bf16)

    t2 = t2_ref[...]
    for ro in range(5):
        y0 = 2 * ro
        s0 = a1_ref[y0:y0 + 5].reshape(560, nb)
        s1 = a1_ref[y0 + 1:y0 + 6].reshape(560, nb)
        o0 = jnp.dot(t2, s0, preferred_element_type=f32)
        o1 = jnp.dot(t2, s1, preferred_element_type=f32)
        m = jnp.maximum(o0, o1) + b2_ref[...]
        m = jnp.maximum(m, 0.0)
        v = m.reshape(5, 2, 16, nb)
        a2_ref[ro] = jnp.maximum(v[:, 0], v[:, 1]).astype(bf16)

    z = a2_ref[...].reshape(400, nb)
    h = jnp.dot(w1_ref[...], z, preferred_element_type=f32) + bf1_ref[...]
    h = jnp.maximum(h, 0.0).astype(bf16)
    h = jnp.dot(w2_ref[...], h, preferred_element_type=f32) + bf2_ref[...]
    h = jnp.maximum(h, 0.0).astype(bf16)
    o_ref[...] = jnp.dot(w3_ref[...], h, preferred_element_type=f32) + bf3_ref[...]


def kernel(w1, b1, w2, b2, fc1_w, fc1_b, fc2_w, fc2_b, fc3_w, fc3_b, x):
    f32 = jnp.float32
    bf16 = jnp.bfloat16
    n = x.shape[0]
    nb = 256 if n % 256 == 0 else 128

    w1r = w1[:, :3, :6].reshape(5, 5, 3, 6)
    v1 = jnp.transpose(w1r, (2, 0, 3, 1))
    t1 = _toeplitz_band(v1, 32, 28)
    t1 = jnp.transpose(t1, (3, 2, 0, 1, 4))
    t1 = jnp.pad(t1, ((0, 0), (0, 2), (0, 0), (0, 0), (0, 0)))
    t1 = t1.reshape(224, 480).astype(bf16)
    b1r = jnp.broadcast_to(jnp.pad(b1[0, :6], (0, 2))[None, :], (28, 8))
    b1r = b1r.reshape(224, 1).astype(f32)

    w2r = w2[:, :6, :16].reshape(5, 5, 6, 16)
    v2 = jnp.transpose(w2r, (0, 2, 3, 1))
    t2 = _toeplitz_band(v2, 14, 10)
    t2 = jnp.transpose(t2, (3, 2, 0, 4, 1))
    t2 = jnp.pad(t2, ((0, 0), (0, 0), (0, 0), (0, 0), (0, 2)))
    t2 = t2.reshape(160, 560).astype(bf16)
    b2r = jnp.broadcast_to(b2[0, :16][None, :], (10, 16)).reshape(160, 1).astype(f32)

    f1 = fc1_w.reshape(5, 5, 128, 128)[:, :, :16, :120]
    w1b = jnp.transpose(f1, (3, 0, 1, 2)).reshape(120, 400)
    w1b = jnp.pad(w1b, ((0, 8), (0, 0))).astype(bf16)
    bf1 = fc1_b.T.astype(f32)
    w2b = fc2_w.T.astype(bf16)
    bf2 = fc2_b.T.astype(f32)
    w3b = fc3_w.T[:16].astype(bf16)
    bf3 = fc3_b[:, :16].T.astype(f32)

    xt = jnp.transpose(x, (1, 2, 3, 0)).astype(bf16)

    def full(a):
        return pl.BlockSpec(a.shape, lambda i: (0,) * a.ndim)

    out = pl.pallas_call(
        functools.partial(_fused_cnn_kernel, nb=nb),
        out_shape=jax.ShapeDtypeStruct((16, n), f32),
        grid=(n // nb,),
        in_specs=[
            pl.BlockSpec((3, 32, 32, nb), lambda i: (0, 0, 0, i)),
            full(t1), full(b1r), full(t2), full(b2r),
            full(w1b), full(bf1), full(w2b), full(bf2), full(w3b), full(bf3),
        ],
        out_specs=pl.BlockSpec((16, nb), lambda i: (0, i)),
        scratch_shapes=[
            pltpu.VMEM((14, 14, 8, nb), bf16),
            pltpu.VMEM((5, 5, 16, nb), bf16),
        ],
        compiler_params=pltpu.CompilerParams(
            dimension_semantics=("parallel",)),
    )(xt, t1, b1r, t2, b2r, w1b, bf1, w2b, bf2, w3b, bf3)

    return out[:10, :].T


# speedup vs baseline: 487.3848x; 1.0865x over previous
import functools

import jax
import jax.numpy as jnp
from jax.experimental import pallas as pl
from jax.experimental.pallas import tpu as pltpu


def _toeplitz_band(v, width, rows):
    k = v.shape[-1]
    period = width + 1
    vp = jnp.pad(v, [(0, 0)] * (v.ndim - 1) + [(0, period - k)])
    t = jnp.tile(vp, (1,) * (v.ndim - 1) + (rows,))[..., : rows * width]
    return t.reshape(v.shape[:-1] + (rows, width))


def _fused_cnn_kernel(x_ref, w_ref, b_ref, o_ref, a1_ref, a2_ref, *, nb):
    f32 = jnp.float32
    bf16 = jnp.bfloat16

    t1 = w_ref[0:224, 0:480]
    b1 = b_ref[0:224]
    for r in range(14):
        y0 = 2 * r
        s0 = x_ref[:, y0:y0 + 5, :, :].reshape(480, nb)
        s1 = x_ref[:, y0 + 1:y0 + 6, :, :].reshape(480, nb)
        o0 = jnp.dot(t1, s0, preferred_element_type=f32)
        o1 = jnp.dot(t1, s1, preferred_element_type=f32)
        m = jnp.maximum(o0, o1) + b1
        m = jnp.maximum(m, 0.0)
        v = m.reshape(14, 2, 8, nb)
        a1_ref[r] = jnp.maximum(v[:, 0], v[:, 1]).astype(bf16)

    t2 = w_ref[224:384, 0:560]
    b2 = b_ref[224:384]
    for ro in range(5):
        y0 = 2 * ro
        s0 = a1_ref[y0:y0 + 5].reshape(560, nb)
        s1 = a1_ref[y0 + 1:y0 + 6].reshape(560, nb)
        o0 = jnp.dot(t2, s0, preferred_element_type=f32)
        o1 = jnp.dot(t2, s1, preferred_element_type=f32)
        m = jnp.maximum(o0, o1) + b2
        m = jnp.maximum(m, 0.0)
        v = m.reshape(5, 2, 16, nb)
        a2_ref[ro] = jnp.maximum(v[:, 0], v[:, 1]).astype(bf16)

    z = a2_ref[...].reshape(400, nb)
    h = jnp.dot(w_ref[384:512, 0:400], z, preferred_element_type=f32) + b_ref[384:512]
    h = jnp.maximum(h, 0.0).astype(bf16)
    h = jnp.dot(w_ref[512:640, 0:128], h, preferred_element_type=f32) + b_ref[512:640]
    h = jnp.maximum(h, 0.0).astype(bf16)
    o_ref[...] = (jnp.dot(w_ref[640:656, 0:128], h, preferred_element_type=f32)
                  + b_ref[640:656])


def kernel(w1, b1, w2, b2, fc1_w, fc1_b, fc2_w, fc2_b, fc3_w, fc3_b, x):
    f32 = jnp.float32
    bf16 = jnp.bfloat16
    n = x.shape[0]
    nb = 512 if n % 512 == 0 else 128

    w1r = w1[:, :3, :6].reshape(5, 5, 3, 6)
    v1 = jnp.transpose(w1r, (2, 0, 3, 1))
    t1 = _toeplitz_band(v1, 32, 28)
    t1 = jnp.transpose(t1, (3, 2, 0, 1, 4))
    t1 = jnp.pad(t1, ((0, 0), (0, 2), (0, 0), (0, 0), (0, 0)))
    t1 = t1.reshape(224, 480)
    b1r = jnp.broadcast_to(jnp.pad(b1[0, :6], (0, 2))[None, :], (28, 8))
    b1r = b1r.reshape(224, 1)

    w2r = w2[:, :6, :16].reshape(5, 5, 6, 16)
    v2 = jnp.transpose(w2r, (0, 2, 3, 1))
    t2 = _toeplitz_band(v2, 14, 10)
    t2 = jnp.transpose(t2, (3, 2, 0, 4, 1))
    t2 = jnp.pad(t2, ((0, 0), (0, 0), (0, 0), (0, 0), (0, 2)))
    t2 = t2.reshape(160, 560)
    b2r = jnp.broadcast_to(b2[0, :16][None, :], (10, 16)).reshape(160, 1)

    f1 = fc1_w.reshape(5, 5, 128, 128)[:, :, :16, :120]
    w1b = jnp.transpose(f1, (3, 0, 1, 2)).reshape(120, 400)
    w1b = jnp.pad(w1b, ((0, 8), (0, 0)))
    w2b = fc2_w.T
    w3b = fc3_w.T[:16]

    def padw(a):
        return jnp.pad(a, ((0, 0), (0, 576 - a.shape[1])))

    wpack = jnp.concatenate(
        [padw(t1), padw(t2), padw(w1b), padw(w2b), padw(w3b)], axis=0
    ).astype(bf16)
    bpack = jnp.concatenate(
        [b1r, b2r, fc1_b.T, fc2_b.T, fc3_b[:, :16].T], axis=0
    ).astype(f32)

    xt = jnp.transpose(x, (1, 2, 3, 0)).astype(bf16)

    out = pl.pallas_call(
        functools.partial(_fused_cnn_kernel, nb=nb),
        out_shape=jax.ShapeDtypeStruct((16, n), f32),
        grid=(n // nb,),
        in_specs=[
            pl.BlockSpec((3, 32, 32, nb), lambda i: (0, 0, 0, i)),
            pl.BlockSpec(wpack.shape, lambda i: (0, 0)),
            pl.BlockSpec(bpack.shape, lambda i: (0, 0)),
        ],
        out_specs=pl.BlockSpec((16, nb), lambda i: (0, i)),
        scratch_shapes=[
            pltpu.VMEM((14, 14, 8, nb), bf16),
            pltpu.VMEM((5, 5, 16, nb), bf16),
        ],
        compiler_params=pltpu.CompilerParams(
            dimension_semantics=("parallel",)),
    )(xt, wpack, bpack)

    return out[:10, :].T


# speedup vs baseline: 495.7720x; 1.0172x over previous
import functools

import jax
import jax.numpy as jnp
from jax.experimental import pallas as pl
from jax.experimental.pallas import tpu as pltpu


def _toeplitz_band(v, width, rows):
    k = v.shape[-1]
    period = width + 1
    vp = jnp.pad(v, [(0, 0)] * (v.ndim - 1) + [(0, period - k)])
    t = jnp.tile(vp, (1,) * (v.ndim - 1) + (rows,))[..., : rows * width]
    return t.reshape(v.shape[:-1] + (rows, width))


def _fused_cnn_kernel(x_ref, w_ref, b_ref, o_ref, a1_ref, a2_ref, *, nb):
    f32 = jnp.float32
    bf16 = jnp.bfloat16

    t1 = w_ref[0:224, 0:480]
    b1 = b_ref[0:224]
    for r in range(14):
        y0 = 2 * r
        s0 = x_ref[:, y0:y0 + 5, :, :].reshape(480, nb)
        s1 = x_ref[:, y0 + 1:y0 + 6, :, :].reshape(480, nb)
        o0 = jnp.dot(t1, s0, preferred_element_type=f32)
        o1 = jnp.dot(t1, s1, preferred_element_type=f32)
        m = jnp.maximum(o0, o1) + b1
        m = jnp.maximum(m, 0.0)
        v = m.reshape(14, 2, 8, nb)
        a1_ref[r] = jnp.maximum(v[:, 0], v[:, 1]).astype(bf16)

    t2 = w_ref[224:384, 0:560]
    b2 = b_ref[224:384]
    for ro in range(5):
        y0 = 2 * ro
        s0 = a1_ref[y0:y0 + 5].reshape(560, nb)
        s1 = a1_ref[y0 + 1:y0 + 6].reshape(560, nb)
        o0 = jnp.dot(t2, s0, preferred_element_type=f32)
        o1 = jnp.dot(t2, s1, preferred_element_type=f32)
        m = jnp.maximum(o0, o1) + b2
        m = jnp.maximum(m, 0.0)
        v = m.reshape(5, 2, 16, nb)
        a2_ref[ro] = jnp.maximum(v[:, 0], v[:, 1]).astype(bf16)

    z = a2_ref[...].reshape(400, nb)
    h = jnp.dot(w_ref[384:512, 0:400], z, preferred_element_type=f32) + b_ref[384:512]
    h = jnp.maximum(h, 0.0).astype(bf16)
    h = jnp.dot(w_ref[512:640, 0:128], h, preferred_element_type=f32) + b_ref[512:640]
    h = jnp.maximum(h, 0.0).astype(bf16)
    o_ref[...] = (jnp.dot(w_ref[640:656, 0:128], h, preferred_element_type=f32)
                  + b_ref[640:656])


def kernel(w1, b1, w2, b2, fc1_w, fc1_b, fc2_w, fc2_b, fc3_w, fc3_b, x):
    f32 = jnp.float32
    bf16 = jnp.bfloat16
    n = x.shape[0]
    nb = 1024 if n % 1024 == 0 else 128

    w1r = w1[:, :3, :6].reshape(5, 5, 3, 6)
    v1 = jnp.transpose(w1r, (2, 0, 3, 1))
    t1 = _toeplitz_band(v1, 32, 28)
    t1 = jnp.transpose(t1, (3, 2, 0, 1, 4))
    t1 = jnp.pad(t1, ((0, 0), (0, 2), (0, 0), (0, 0), (0, 0)))
    t1 = t1.reshape(224, 480)
    b1r = jnp.broadcast_to(jnp.pad(b1[0, :6], (0, 2))[None, :], (28, 8))
    b1r = b1r.reshape(224, 1)

    w2r = w2[:, :6, :16].reshape(5, 5, 6, 16)
    v2 = jnp.transpose(w2r, (0, 2, 3, 1))
    t2 = _toeplitz_band(v2, 14, 10)
    t2 = jnp.transpose(t2, (3, 2, 0, 4, 1))
    t2 = jnp.pad(t2, ((0, 0), (0, 0), (0, 0), (0, 0), (0, 2)))
    t2 = t2.reshape(160, 560)
    b2r = jnp.broadcast_to(b2[0, :16][None, :], (10, 16)).reshape(160, 1)

    f1 = fc1_w.reshape(5, 5, 128, 128)[:, :, :16, :120]
    w1b = jnp.transpose(f1, (3, 0, 1, 2)).reshape(120, 400)
    w1b = jnp.pad(w1b, ((0, 8), (0, 0)))
    w2b = fc2_w.T
    w3b = fc3_w.T[:16]

    def padw(a):
        return jnp.pad(a, ((0, 0), (0, 576 - a.shape[1])))

    wpack = jnp.concatenate(
        [padw(t1), padw(t2), padw(w1b), padw(w2b), padw(w3b)], axis=0
    ).astype(bf16)
    bpack = jnp.concatenate(
        [b1r, b2r, fc1_b.T, fc2_b.T, fc3_b[:, :16].T], axis=0
    ).astype(f32)

    xt = jnp.transpose(x, (1, 2, 3, 0)).astype(bf16)

    out = pl.pallas_call(
        functools.partial(_fused_cnn_kernel, nb=nb),
        out_shape=jax.ShapeDtypeStruct((16, n), f32),
        grid=(n // nb,),
        in_specs=[
            pl.BlockSpec((3, 32, 32, nb), lambda i: (0, 0, 0, i)),
            pl.BlockSpec(wpack.shape, lambda i: (0, 0)),
            pl.BlockSpec(bpack.shape, lambda i: (0, 0)),
        ],
        out_specs=pl.BlockSpec((16, nb), lambda i: (0, i)),
        scratch_shapes=[
            pltpu.VMEM((14, 14, 8, nb), bf16),
            pltpu.VMEM((5, 5, 16, nb), bf16),
        ],
        compiler_params=pltpu.CompilerParams(
            dimension_semantics=("parallel",)),
    )(xt, wpack, bpack)

    return out[:10, :].T


# speedup vs baseline: 529.7582x; 1.0686x over previous
import functools

import jax
import jax.numpy as jnp
from jax.experimental import pallas as pl
from jax.experimental.pallas import tpu as pltpu


import numpy as np


def _band_const(rows, width):
    c = np.zeros((25, 5, rows, width), np.float32)
    for t in range(25):
        dy, dx = divmod(t, 5)
        for x in range(rows):
            c[t, dy, x, x + dx] = 1.0
    return c.reshape(25, 5 * rows * width)


def _fused_cnn_kernel(x_ref, w_ref, b_ref, o_ref, a1_ref, a2_ref, *, nb):
    f32 = jnp.float32
    bf16 = jnp.bfloat16

    t1 = w_ref[0:224, 0:480]
    b1 = b_ref[0:224]
    for r in range(14):
        y0 = 2 * r
        s0 = x_ref[:, y0:y0 + 5, :, :].reshape(480, nb)
        s1 = x_ref[:, y0 + 1:y0 + 6, :, :].reshape(480, nb)
        o0 = jnp.dot(t1, s0, preferred_element_type=f32)
        o1 = jnp.dot(t1, s1, preferred_element_type=f32)
        m = jnp.maximum(o0, o1) + b1
        m = jnp.maximum(m, 0.0)
        v = m.reshape(14, 2, 8, nb)
        a1_ref[r] = jnp.maximum(v[:, 0], v[:, 1]).astype(bf16)

    t2 = w_ref[224:384, 0:560]
    b2 = b_ref[224:384]
    for ro in range(5):
        y0 = 2 * ro
        s0 = a1_ref[y0:y0 + 5].reshape(560, nb)
        s1 = a1_ref[y0 + 1:y0 + 6].reshape(560, nb)
        o0 = jnp.dot(t2, s0, preferred_element_type=f32)
        o1 = jnp.dot(t2, s1, preferred_element_type=f32)
        m = jnp.maximum(o0, o1) + b2
        m = jnp.maximum(m, 0.0)
        v = m.reshape(5, 2, 16, nb)
        a2_ref[ro] = jnp.maximum(v[:, 0], v[:, 1]).astype(bf16)

    z = a2_ref[...].reshape(400, nb)
    h = jnp.dot(w_ref[384:512, 0:400], z, preferred_element_type=f32) + b_ref[384:512]
    h = jnp.maximum(h, 0.0).astype(bf16)
    h = jnp.dot(w_ref[512:640, 0:128], h, preferred_element_type=f32) + b_ref[512:640]
    h = jnp.maximum(h, 0.0).astype(bf16)
    o_ref[...] = (jnp.dot(w_ref[640:656, 0:128], h, preferred_element_type=f32)
                  + b_ref[640:656])


def kernel(w1, b1, w2, b2, fc1_w, fc1_b, fc2_w, fc2_b, fc3_w, fc3_b, x):
    f32 = jnp.float32
    bf16 = jnp.bfloat16
    n = x.shape[0]
    nb = 1024 if n % 1024 == 0 else 128

    c1 = _band_const(28, 32)
    t1 = jax.lax.dot_general(w1[:, :3, :6].reshape(25, 18), c1,
                             (((0,), (0,)), ((), ())))
    t1 = t1.reshape(3, 6, 5, 28, 32)
    t1 = jnp.transpose(t1, (3, 1, 0, 2, 4))
    t1 = jnp.pad(t1, ((0, 0), (0, 2), (0, 0), (0, 0), (0, 0)))
    t1 = t1.reshape(224, 480)
    b1r = jnp.broadcast_to(jnp.pad(b1[0, :6], (0, 2))[None, :], (28, 8))
    b1r = b1r.reshape(224, 1)

    c2 = _band_const(10, 14)
    t2 = jax.lax.dot_general(w2[:, :6, :16].reshape(25, 96), c2,
                             (((0,), (0,)), ((), ())))
    t2 = t2.reshape(6, 16, 5, 10, 14)
    t2 = jnp.transpose(t2, (3, 1, 2, 4, 0))
    t2 = jnp.pad(t2, ((0, 0), (0, 0), (0, 0), (0, 0), (0, 2)))
    t2 = t2.reshape(160, 560)
    b2r = jnp.broadcast_to(b2[0, :16][None, :], (10, 16)).reshape(160, 1)

    f1 = fc1_w.reshape(5, 5, 128, 128)[:, :, :16, :120]
    w1b = jnp.transpose(f1, (3, 0, 1, 2)).reshape(120, 400)
    w1b = jnp.pad(w1b, ((0, 8), (0, 0)))
    w2b = fc2_w.T
    w3b = fc3_w.T[:16]

    def padw(a):
        return jnp.pad(a, ((0, 0), (0, 576 - a.shape[1])))

    wpack = jnp.concatenate(
        [padw(t1), padw(t2), padw(w1b), padw(w2b), padw(w3b)], axis=0
    ).astype(bf16)
    bpack = jnp.concatenate(
        [b1r, b2r, fc1_b.T, fc2_b.T, fc3_b[:, :16].T], axis=0
    ).astype(f32)

    xt = jnp.transpose(x, (1, 2, 3, 0)).astype(bf16)

    out = pl.pallas_call(
        functools.partial(_fused_cnn_kernel, nb=nb),
        out_shape=jax.ShapeDtypeStruct((16, n), f32),
        grid=(n // nb,),
        in_specs=[
            pl.BlockSpec((3, 32, 32, nb), lambda i: (0, 0, 0, i)),
            pl.BlockSpec(wpack.shape, lambda i: (0, 0)),
            pl.BlockSpec(bpack.shape, lambda i: (0, 0)),
        ],
        out_specs=pl.BlockSpec((16, nb), lambda i: (0, i)),
        scratch_shapes=[
            pltpu.VMEM((14, 14, 8, nb), bf16),
            pltpu.VMEM((5, 5, 16, nb), bf16),
        ],
        compiler_params=pltpu.CompilerParams(
            dimension_semantics=("parallel",)),
    )(xt, wpack, bpack)

    return out[:10, :].T


# speedup vs baseline: 530.1565x; 1.0008x over previous
import functools

import jax
import jax.numpy as jnp
from jax.experimental import pallas as pl
from jax.experimental.pallas import tpu as pltpu


import numpy as np


def _band_const(rows, width):
    c = np.zeros((25, 5, rows, width), np.float32)
    for t in range(25):
        dy, dx = divmod(t, 5)
        for x in range(rows):
            c[t, dy, x, x + dx] = 1.0
    return c.reshape(25, 5 * rows * width)


def _fused_cnn_kernel(x_ref, w_ref, b_ref, o_ref, a1_ref, a2_ref, *, nb):
    f32 = jnp.float32
    bf16 = jnp.bfloat16

    t1 = w_ref[0:224, 0:480]
    b1 = b_ref[0:224]
    for r in range(14):
        y0 = 2 * r
        s0 = x_ref[:, y0:y0 + 5, :, :].reshape(480, nb)
        s1 = x_ref[:, y0 + 1:y0 + 6, :, :].reshape(480, nb)
        o0 = jnp.dot(t1, s0, preferred_element_type=f32)
        o1 = jnp.dot(t1, s1, preferred_element_type=f32)
        m = jnp.maximum(o0, o1) + b1
        m = jnp.maximum(m, 0.0)
        v = m.reshape(14, 2, 8, nb)
        a1_ref[r] = jnp.maximum(v[:, 0], v[:, 1]).astype(bf16)

    t2 = w_ref[224:384, 0:560]
    b2 = b_ref[224:384]
    for ro in range(5):
        y0 = 2 * ro
        s0 = a1_ref[y0:y0 + 5].reshape(560, nb)
        s1 = a1_ref[y0 + 1:y0 + 6].reshape(560, nb)
        o0 = jnp.dot(t2, s0, preferred_element_type=f32)
        o1 = jnp.dot(t2, s1, preferred_element_type=f32)
        m = jnp.maximum(o0, o1) + b2
        m = jnp.maximum(m, 0.0)
        v = m.reshape(5, 2, 16, nb)
        a2_ref[ro] = jnp.maximum(v[:, 0], v[:, 1]).astype(bf16)

    z = a2_ref[...].reshape(400, nb)
    h = jnp.dot(w_ref[384:512, 0:400], z, preferred_element_type=f32) + b_ref[384:512]
    h = jnp.maximum(h, 0.0).astype(bf16)
    h = jnp.dot(w_ref[512:640, 0:128], h, preferred_element_type=f32) + b_ref[512:640]
    h = jnp.maximum(h, 0.0).astype(bf16)
    o_ref[...] = (jnp.dot(w_ref[640:656, 0:128], h, preferred_element_type=f32)
                  + b_ref[640:656])


def kernel(w1, b1, w2, b2, fc1_w, fc1_b, fc2_w, fc2_b, fc3_w, fc3_b, x):
    f32 = jnp.float32
    bf16 = jnp.bfloat16
    n = x.shape[0]
    nb = 1024 if n % 1024 == 0 else 128

    c1 = _band_const(28, 32)
    t1 = jax.lax.dot_general(w1[:, :3, :6].reshape(25, 18), c1,
                             (((0,), (0,)), ((), ())))
    t1 = t1.reshape(3, 6, 5, 28, 32)
    t1 = jnp.transpose(t1, (3, 1, 0, 2, 4))
    t1 = jnp.pad(t1, ((0, 0), (0, 2), (0, 0), (0, 0), (0, 0)))
    t1 = t1.reshape(224, 480)
    b1r = jnp.broadcast_to(jnp.pad(b1[0, :6], (0, 2))[None, :], (28, 8))
    b1r = b1r.reshape(224, 1)

    c2 = _band_const(10, 14)
    t2 = jax.lax.dot_general(w2[:, :6, :16].reshape(25, 96), c2,
                             (((0,), (0,)), ((), ())))
    t2 = t2.reshape(6, 16, 5, 10, 14)
    t2 = jnp.transpose(t2, (3, 1, 2, 4, 0))
    t2 = jnp.pad(t2, ((0, 0), (0, 0), (0, 0), (0, 0), (0, 2)))
    t2 = t2.reshape(160, 560)
    b2r = jnp.broadcast_to(b2[0, :16][None, :], (10, 16)).reshape(160, 1)

    f1 = fc1_w.reshape(5, 5, 128, 128)[:, :, :16, :120]
    w1b = jnp.transpose(f1, (3, 0, 1, 2)).reshape(120, 400)
    w1b = jnp.pad(w1b, ((0, 8), (0, 0)))
    w2b = fc2_w.T
    w3b = fc3_w.T[:16]

    def padw(a):
        return jnp.pad(a, ((0, 0), (0, 576 - a.shape[1])))

    wpack = jnp.concatenate(
        [padw(t1), padw(t2), padw(w1b), padw(w2b), padw(w3b)], axis=0
    ).astype(bf16)
    bpack = jnp.concatenate(
        [b1r, b2r, fc1_b.T, fc2_b.T, fc3_b[:, :16].T], axis=0
    ).astype(f32)

    xt = jnp.transpose(x, (1, 2, 3, 0)).astype(bf16)

    out = pl.pallas_call(
        functools.partial(_fused_cnn_kernel, nb=nb),
        out_shape=jax.ShapeDtypeStruct((16, n), f32),
        grid=(n // nb,),
        in_specs=[
            pl.BlockSpec((3, 32, 32, nb), lambda i: (0, 0, 0, i)),
            pl.BlockSpec(wpack.shape, lambda i: (0, 0)),
            pl.BlockSpec(bpack.shape, lambda i: (0, 0)),
        ],
        out_specs=pl.BlockSpec((16, nb), lambda i: (0, i)),
        scratch_shapes=[
            pltpu.VMEM((14, 14, 8, nb), bf16),
            pltpu.VMEM((5, 5, 16, nb), bf16),
        ],
        compiler_params=pltpu.CompilerParams(
            dimension_semantics=("parallel",),
            allow_input_fusion=[True, False, False]),
    )(xt, wpack, bpack)

    return out[:10, :].T
